# Initial kernel scaffold; baseline (speedup 1.0000x reference)
import functools

import jax
import jax.numpy as jnp
from jax import lax
from jax.experimental import pallas as pl
from jax.experimental.pallas import tpu as pltpu
from jax.experimental.pallas import tpu_sc as plsc

D = 32
INPUT_DIM = 128
NUM_TABLES = 4


def _sc_gather(n_rows, chunk):
    info = plsc.get_sparse_core_info()
    nc, ns = info.num_cores, info.num_subcores
    nw = nc * ns
    rows_per_w = n_rows // nw
    assert n_rows % nw == 0 and rows_per_w % chunk == 0
    n_chunks = rows_per_w // chunk

    mesh = plsc.VectorSubcoreMesh(core_axis_name="c", subcore_axis_name="s")

    @functools.partial(
        pl.kernel,
        mesh=mesh,
        out_type=[jax.ShapeDtypeStruct((n_rows, D), jnp.float32)
                  for _ in range(NUM_TABLES)],
        scratch_types=[
            pltpu.VMEM((chunk,), jnp.int32),
            pltpu.VMEM((chunk, D), jnp.float32),
            pltpu.SemaphoreType.DMA,
        ],
    )
    def k(tab0, tab1, tab2, tab3, idx0, idx1, idx2, idx3,
          out0, out1, out2, out3, idx_v, rows_v, sem):
        wid = lax.axis_index("s") * nc + lax.axis_index("c")
        tabs = (tab0, tab1, tab2, tab3)
        idxs = (idx0, idx1, idx2, idx3)
        outs = (out0, out1, out2, out3)

        def body(ci, _):
            base = wid * rows_per_w + ci * chunk
            for t in range(NUM_TABLES):
                pltpu.sync_copy(idxs[t].at[pl.ds(base, chunk)], idx_v)
                pltpu.async_copy(tabs[t].at[idx_v], rows_v, sem).wait()
                pltpu.sync_copy(rows_v, outs[t].at[pl.ds(base, chunk)])
            return ()

        lax.fori_loop(0, n_chunks, body, ())

    return k


def _tc_project(gi, gu, gt, gn, W, b, block_n):
    n_rows = gi.shape[0]
    grid = (n_rows // block_n,)

    def body(gi_ref, gu_ref, gt_ref, gn_ref, w_ref, b_ref, o_ref):
        w = w_ref[...]
        acc = jax.lax.dot_general(
            gi_ref[...], w[0:D, :], (((1,), (0,)), ((), ())),
            preferred_element_type=jnp.float32)
        acc += jax.lax.dot_general(
            gu_ref[...], w[D:2 * D, :], (((1,), (0,)), ((), ())),
            preferred_element_type=jnp.float32)
        acc += jax.lax.dot_general(
            gt_ref[...], w[2 * D:3 * D, :], (((1,), (0,)), ((), ())),
            preferred_element_type=jnp.float32)
        acc += jax.lax.dot_general(
            gn_ref[...], w[3 * D:4 * D, :], (((1,), (0,)), ((), ())),
            preferred_element_type=jnp.float32)
        o_ref[...] = acc + b_ref[...]

    in_block = pl.BlockSpec((block_n, D), lambda i: (i, 0))
    return pl.pallas_call(
        body,
        grid=grid,
        in_specs=[in_block, in_block, in_block, in_block,
                  pl.BlockSpec((4 * D, INPUT_DIM), lambda i: (0, 0)),
                  pl.BlockSpec((1, INPUT_DIM), lambda i: (0, 0))],
        out_specs=pl.BlockSpec((block_n, INPUT_DIM), lambda i: (i, 0)),
        out_shape=jax.ShapeDtypeStruct((n_rows, INPUT_DIM), jnp.float32),
    )(gi, gu, gt, gn, W, b.reshape(1, INPUT_DIM))


def kernel(item, user, tag, interaction, emb_item, emb_user, emb_tag,
           emb_interaction, W, b):
    B, L = item.shape
    n_rows = B * L

    idx_item = item.reshape(-1).astype(jnp.int32)
    idx_user = user.reshape(-1).astype(jnp.int32)
    idx_tag = tag.reshape(-1).astype(jnp.int32)
    idx_int = interaction.reshape(-1).astype(jnp.int32)

    gather = _sc_gather(n_rows, 1600)
    gi, gu, gt, gn = gather(emb_item, emb_user, emb_tag, emb_interaction,
                            idx_item, idx_user, idx_tag, idx_int)

    out = _tc_project(gi, gu, gt, gn, W, b, block_n=2048)
    return out.reshape(B, L, INPUT_DIM)


# speedup vs baseline: 2.4072x; 2.4072x over previous
import functools

import jax
import jax.numpy as jnp
from jax import lax
from jax.experimental import pallas as pl
from jax.experimental.pallas import tpu as pltpu
from jax.experimental.pallas import tpu_sc as plsc

D = 32
INPUT_DIM = 128
NUM_TABLES = 4


def _sc_gather(n_rows, chunk):
    info = plsc.get_sparse_core_info()
    nc, ns = info.num_cores, info.num_subcores
    nw = nc * ns
    rows_per_w = n_rows // nw
    assert n_rows % nw == 0 and rows_per_w % chunk == 0
    n_chunks = rows_per_w // chunk

    mesh = plsc.VectorSubcoreMesh(core_axis_name="c", subcore_axis_name="s")

    @functools.partial(
        pl.kernel,
        mesh=mesh,
        compiler_params=pltpu.CompilerParams(use_tc_tiling_on_sc=False),
        out_type=[jax.ShapeDtypeStruct((n_rows, D), jnp.float32)
                  for _ in range(NUM_TABLES)],
        scratch_types=[
            pltpu.VMEM((chunk,), jnp.int32),
            pltpu.VMEM((chunk, D), jnp.float32),
            pltpu.SemaphoreType.DMA,
        ],
    )
    def k(tab0, tab1, tab2, tab3, idx0, idx1, idx2, idx3,
          out0, out1, out2, out3, idx_v, rows_v, sem):
        wid = lax.axis_index("s") * nc + lax.axis_index("c")
        tabs = (tab0, tab1, tab2, tab3)
        idxs = (idx0, idx1, idx2, idx3)
        outs = (out0, out1, out2, out3)

        def body(ci, _):
            base = wid * rows_per_w + ci * chunk
            for t in range(NUM_TABLES):
                pltpu.sync_copy(idxs[t].at[pl.ds(base, chunk)], idx_v)
                pltpu.async_copy(tabs[t].at[idx_v], rows_v, sem).wait()
                pltpu.sync_copy(rows_v, outs[t].at[pl.ds(base, chunk)])
            return ()

        lax.fori_loop(0, n_chunks, body, ())

    return k


def _tc_project(gi, gu, gt, gn, W, b, block_n):
    n_rows = gi.shape[0]
    grid = (n_rows // block_n,)

    def body(gi_ref, gu_ref, gt_ref, gn_ref, w_ref, b_ref, o_ref):
        w = w_ref[...]
        acc = jax.lax.dot_general(
            gi_ref[...], w[0:D, :], (((1,), (0,)), ((), ())),
            preferred_element_type=jnp.float32)
        acc += jax.lax.dot_general(
            gu_ref[...], w[D:2 * D, :], (((1,), (0,)), ((), ())),
            preferred_element_type=jnp.float32)
        acc += jax.lax.dot_general(
            gt_ref[...], w[2 * D:3 * D, :], (((1,), (0,)), ((), ())),
            preferred_element_type=jnp.float32)
        acc += jax.lax.dot_general(
            gn_ref[...], w[3 * D:4 * D, :], (((1,), (0,)), ((), ())),
            preferred_element_type=jnp.float32)
        o_ref[...] = acc + b_ref[...]

    in_block = pl.BlockSpec((block_n, D), lambda i: (i, 0))
    return pl.pallas_call(
        body,
        grid=grid,
        in_specs=[in_block, in_block, in_block, in_block,
                  pl.BlockSpec((4 * D, INPUT_DIM), lambda i: (0, 0)),
                  pl.BlockSpec((1, INPUT_DIM), lambda i: (0, 0))],
        out_specs=pl.BlockSpec((block_n, INPUT_DIM), lambda i: (i, 0)),
        out_shape=jax.ShapeDtypeStruct((n_rows, INPUT_DIM), jnp.float32),
    )(gi, gu, gt, gn, W, b.reshape(1, INPUT_DIM))


def kernel(item, user, tag, interaction, emb_item, emb_user, emb_tag,
           emb_interaction, W, b):
    B, L = item.shape
    n_rows = B * L

    idx_item = item.reshape(-1).astype(jnp.int32)
    idx_user = user.reshape(-1).astype(jnp.int32)
    idx_tag = tag.reshape(-1).astype(jnp.int32)
    idx_int = interaction.reshape(-1).astype(jnp.int32)

    gather = _sc_gather(n_rows, 1600)
    gi, gu, gt, gn = gather(emb_item, emb_user, emb_tag, emb_interaction,
                            idx_item, idx_user, idx_tag, idx_int)

    out = _tc_project(gi, gu, gt, gn, W, b, block_n=2048)
    return out.reshape(B, L, INPUT_DIM)
